# baseline (speedup 1.0000x reference)
import functools

import jax
import jax.numpy as jnp
from jax import lax
from jax.experimental import pallas as pl
from jax.experimental.pallas import tpu as pltpu
from jax.experimental.pallas import tpu_sc as plsc

_BATCH = 1024
_N = 100000
_F = 32
_INV_TEMP = 20.0
_SHIFT = 20.0

_BLK = 4096
_NBLK = 24
_TBLK = 2048
_TIDX = 48

_NC = 2
_NS = 16
_NW = _NC * _NS
_BPW = _BATCH // _NW
_REG = 16


def _sc_gather128(idx, table, shift, out_dtype):
    mesh = plsc.VectorSubcoreMesh(core_axis_name="c", subcore_axis_name="s")

    @functools.partial(
        pl.kernel,
        mesh=mesh,
        out_type=jax.ShapeDtypeStruct((_BATCH, 128), out_dtype),
        scratch_types=[
            pltpu.VMEM((_BPW,), jnp.int32),
            pltpu.VMEM((_BPW,), jnp.int32),
            pltpu.VMEM((_BPW, 128), out_dtype),
            pltpu.SemaphoreType.DMA,
        ],
    )
    def k(idx_hbm, tab_hbm, out_hbm, idx_v, q_v, rows_v, sem):
        wid = lax.axis_index("s") * _NC + lax.axis_index("c")
        base = wid * _BPW
        pltpu.sync_copy(idx_hbm.at[pl.ds(base, _BPW)], idx_v)

        @pl.loop(0, _BPW, step=_REG)
        def _(j):
            sl = pl.ds(j, _REG)
            q_v.at[sl][...] = lax.shift_right_logical(idx_v.at[sl][...], shift)

        pltpu.async_copy(tab_hbm.at[q_v], rows_v, sem).wait()
        pltpu.sync_copy(rows_v, out_hbm.at[pl.ds(base, _BPW)])

    return k(idx, table)


def _extract_t(lab, idx):
    r = idx & 127
    col = lax.broadcasted_iota(jnp.int32, (_BATCH, 128), 1)
    return jnp.sum(jnp.where(col == r, lab, 0), axis=1, keepdims=True)


def _lane_chunk_sum(e, width):
    s = e[:, 0:128]
    for c in range(1, width // 128):
        s = s + e[:, 128 * c:128 * (c + 1)]
    return s


def _pick_chunks(logits, p, width):
    pcb = lax.shift_right_arithmetic(p, 7) + jnp.zeros((_BATCH, 128),
                                                       jnp.int32)
    pk = jnp.where(pcb == 0, logits[:, 0:128], 0.0)
    for c in range(1, width // 128):
        pk = jnp.where(pcb == c, logits[:, 128 * c:128 * (c + 1)], pk)
    return pk


def _tc_main_body(inp_ref, feat_ref, lab_ref, idx_ref, acc_ref, pick_ref,
                  t_ref):
    i = pl.program_id(0)

    @pl.when(i == 0)
    def _():
        t_ref[...] = _extract_t(lab_ref[...], idx_ref[...])

    inp = (inp_ref[...] * _INV_TEMP).astype(jnp.bfloat16)
    logits = lax.dot_general(
        inp, feat_ref[...].astype(jnp.bfloat16),
        dimension_numbers=(((0,), (0,)), ((), ())),
        preferred_element_type=jnp.float32,
    )
    s = _lane_chunk_sum(jnp.exp(logits - _SHIFT), _BLK)
    pk = _pick_chunks(logits, t_ref[...] - i * _BLK, _BLK)

    @pl.when(i == 0)
    def _():
        acc_ref[...] = s
        pick_ref[...] = pk

    @pl.when(i > 0)
    def _():
        acc_ref[...] += s
        pick_ref[...] += pk


def _tc_main(inputs, featT, labrows, idx2d):
    return pl.pallas_call(
        _tc_main_body,
        grid=(_NBLK,),
        in_specs=[
            pl.BlockSpec((_F, _BATCH), lambda i: (0, 0)),
            pl.BlockSpec((_F, _BLK), lambda i: (0, i)),
            pl.BlockSpec((_BATCH, 128), lambda i: (0, 0)),
            pl.BlockSpec((_BATCH, 1), lambda i: (0, 0)),
        ],
        out_specs=[
            pl.BlockSpec((_BATCH, 128), lambda i: (0, 0)),
            pl.BlockSpec((_BATCH, 128), lambda i: (0, 0)),
        ],
        out_shape=[
            jax.ShapeDtypeStruct((_BATCH, 128), jnp.float32),
            jax.ShapeDtypeStruct((_BATCH, 128), jnp.float32),
        ],
        scratch_shapes=[pltpu.VMEM((_BATCH, 1), jnp.int32)],
    )(inputs, featT, labrows, idx2d)


def _tc_finish_body(inp_ref, feat_ref, lab_ref, idx_ref, acc_ref, pick_ref,
                    out_ref):
    t = _extract_t(lab_ref[...], idx_ref[...])

    inp = (inp_ref[...] * _INV_TEMP).astype(jnp.bfloat16)
    logits = lax.dot_general(
        inp, feat_ref[...].astype(jnp.bfloat16),
        dimension_numbers=(((0,), (0,)), ((), ())),
        preferred_element_type=jnp.float32,
    )
    col = _TIDX * _TBLK + lax.broadcasted_iota(jnp.int32, (_BATCH, _TBLK), 1)
    e = jnp.where(col < _N, jnp.exp(logits - _SHIFT), 0.0)
    se = jnp.sum(acc_ref[...] + _lane_chunk_sum(e, _TBLK), axis=1,
                 keepdims=True)
    lse = jnp.log(se) + _SHIFT

    pk = pick_ref[...] + _pick_chunks(logits, t - _TIDX * _TBLK, _TBLK)
    lane = lax.broadcasted_iota(jnp.int32, (_BATCH, 128), 1)
    picked = jnp.sum(jnp.where(lane == (t & 127), pk, 0.0), axis=1,
                     keepdims=True)
    out_ref[...] = jnp.mean(lse - picked)[None, None]


def _tc_finish(inputs, featT, labrows, idx2d, acc, pick):
    return pl.pallas_call(
        _tc_finish_body,
        grid=(1,),
        in_specs=[
            pl.BlockSpec((_F, _BATCH), lambda i: (0, 0)),
            pl.BlockSpec((_F, _TBLK), lambda i: (0, _TIDX)),
            pl.BlockSpec((_BATCH, 128), lambda i: (0, 0)),
            pl.BlockSpec((_BATCH, 1), lambda i: (0, 0)),
            pl.BlockSpec((_BATCH, 128), lambda i: (0, 0)),
            pl.BlockSpec((_BATCH, 128), lambda i: (0, 0)),
        ],
        out_specs=pl.BlockSpec((1, 1), lambda i: (0, 0)),
        out_shape=jax.ShapeDtypeStruct((1, 1), jnp.float32),
    )(inputs, featT, labrows, idx2d, acc, pick)


def kernel(inputs, indices, features, labels):
    labpad = jnp.pad(labels, (0, 782 * 128 - _N)).reshape(782, 128)
    featT = features.T
    inpT = inputs.T
    idx2d = indices.reshape(_BATCH, 1)
    labrows = _sc_gather128(indices, labpad, 7, jnp.int32)
    acc, pick = _tc_main(inpT, featT, labrows, idx2d)
    loss = _tc_finish(inpT, featT, labrows, idx2d, acc, pick)
    return loss[0, 0]


# speedup vs baseline: 1.0382x; 1.0382x over previous
import functools

import jax
import jax.numpy as jnp
from jax import lax
from jax.experimental import pallas as pl
from jax.experimental.pallas import tpu as pltpu
from jax.experimental.pallas import tpu_sc as plsc

_BATCH = 1024
_N = 100000
_F = 32
_INV_TEMP = 20.0
_SHIFT = 20.0

_BLK = 8192
_NBLK = 12
_TBLK = 2048
_TIDX = 48

_NC = 2
_NS = 16
_NW = _NC * _NS
_BPW = _BATCH // _NW
_REG = 16


def _sc_gather128(idx, table, shift, out_dtype):
    mesh = plsc.VectorSubcoreMesh(core_axis_name="c", subcore_axis_name="s")

    @functools.partial(
        pl.kernel,
        mesh=mesh,
        out_type=jax.ShapeDtypeStruct((_BATCH, 128), out_dtype),
        scratch_types=[
            pltpu.VMEM((_BPW,), jnp.int32),
            pltpu.VMEM((_BPW,), jnp.int32),
            pltpu.VMEM((_BPW, 128), out_dtype),
            pltpu.SemaphoreType.DMA,
        ],
    )
    def k(idx_hbm, tab_hbm, out_hbm, idx_v, q_v, rows_v, sem):
        wid = lax.axis_index("s") * _NC + lax.axis_index("c")
        base = wid * _BPW
        pltpu.sync_copy(idx_hbm.at[pl.ds(base, _BPW)], idx_v)

        @pl.loop(0, _BPW, step=_REG)
        def _(j):
            sl = pl.ds(j, _REG)
            q_v.at[sl][...] = lax.shift_right_logical(idx_v.at[sl][...], shift)

        pltpu.async_copy(tab_hbm.at[q_v], rows_v, sem).wait()
        pltpu.sync_copy(rows_v, out_hbm.at[pl.ds(base, _BPW)])

    return k(idx, table)


def _extract_t(lab, idx):
    r = idx & 127
    col = lax.broadcasted_iota(jnp.int32, (_BATCH, 128), 1)
    return jnp.sum(jnp.where(col == r, lab, 0), axis=1, keepdims=True)


def _lane_chunk_sum(e, width):
    s = e[:, 0:128]
    for c in range(1, width // 128):
        s = s + e[:, 128 * c:128 * (c + 1)]
    return s


def _pick_chunks(logits, p, width):
    pcb = lax.shift_right_arithmetic(p, 7) + jnp.zeros((_BATCH, 128),
                                                       jnp.int32)
    pk = jnp.where(pcb == 0, logits[:, 0:128], 0.0)
    for c in range(1, width // 128):
        pk = jnp.where(pcb == c, logits[:, 128 * c:128 * (c + 1)], pk)
    return pk


def _tc_main_body(inp_ref, feat_ref, lab_ref, idx_ref, acc_ref, pick_ref,
                  t_ref):
    i = pl.program_id(0)

    @pl.when(i == 0)
    def _():
        t_ref[...] = _extract_t(lab_ref[...], idx_ref[...])

    inp = (inp_ref[...] * _INV_TEMP).astype(jnp.bfloat16)
    logits = lax.dot_general(
        inp, feat_ref[...].astype(jnp.bfloat16),
        dimension_numbers=(((1,), (0,)), ((), ())),
        preferred_element_type=jnp.float32,
    )
    s = _lane_chunk_sum(jnp.exp(logits - _SHIFT), _BLK)
    pk = _pick_chunks(logits, t_ref[...] - i * _BLK, _BLK)

    @pl.when(i == 0)
    def _():
        acc_ref[...] = s
        pick_ref[...] = pk

    @pl.when(i > 0)
    def _():
        acc_ref[...] += s
        pick_ref[...] += pk


def _tc_main(inputs, featT, labrows, idx2d):
    return pl.pallas_call(
        _tc_main_body,
        grid=(_NBLK,),
        in_specs=[
            pl.BlockSpec((_BATCH, _F), lambda i: (0, 0)),
            pl.BlockSpec((_F, _BLK), lambda i: (0, i)),
            pl.BlockSpec((_BATCH, 128), lambda i: (0, 0)),
            pl.BlockSpec((_BATCH, 1), lambda i: (0, 0)),
        ],
        out_specs=[
            pl.BlockSpec((_BATCH, 128), lambda i: (0, 0)),
            pl.BlockSpec((_BATCH, 128), lambda i: (0, 0)),
        ],
        out_shape=[
            jax.ShapeDtypeStruct((_BATCH, 128), jnp.float32),
            jax.ShapeDtypeStruct((_BATCH, 128), jnp.float32),
        ],
        scratch_shapes=[pltpu.VMEM((_BATCH, 1), jnp.int32)],
    )(inputs, featT, labrows, idx2d)


def _tc_finish_body(inp_ref, feat_ref, lab_ref, idx_ref, acc_ref, pick_ref,
                    out_ref):
    t = _extract_t(lab_ref[...], idx_ref[...])

    inp = (inp_ref[...] * _INV_TEMP).astype(jnp.bfloat16)
    logits = lax.dot_general(
        inp, feat_ref[...].astype(jnp.bfloat16),
        dimension_numbers=(((1,), (0,)), ((), ())),
        preferred_element_type=jnp.float32,
    )
    col = _TIDX * _TBLK + lax.broadcasted_iota(jnp.int32, (_BATCH, _TBLK), 1)
    e = jnp.where(col < _N, jnp.exp(logits - _SHIFT), 0.0)
    se = jnp.sum(acc_ref[...] + _lane_chunk_sum(e, _TBLK), axis=1,
                 keepdims=True)
    lse = jnp.log(se) + _SHIFT

    pk = pick_ref[...] + _pick_chunks(logits, t - _TIDX * _TBLK, _TBLK)
    lane = lax.broadcasted_iota(jnp.int32, (_BATCH, 128), 1)
    picked = jnp.sum(jnp.where(lane == (t & 127), pk, 0.0), axis=1,
                     keepdims=True)
    out_ref[...] = jnp.mean(lse - picked)[None, None]


def _tc_finish(inputs, featT, labrows, idx2d, acc, pick):
    return pl.pallas_call(
        _tc_finish_body,
        grid=(1,),
        in_specs=[
            pl.BlockSpec((_BATCH, _F), lambda i: (0, 0)),
            pl.BlockSpec((_F, _TBLK), lambda i: (0, _TIDX)),
            pl.BlockSpec((_BATCH, 128), lambda i: (0, 0)),
            pl.BlockSpec((_BATCH, 1), lambda i: (0, 0)),
            pl.BlockSpec((_BATCH, 128), lambda i: (0, 0)),
            pl.BlockSpec((_BATCH, 128), lambda i: (0, 0)),
        ],
        out_specs=pl.BlockSpec((1, 1), lambda i: (0, 0)),
        out_shape=jax.ShapeDtypeStruct((1, 1), jnp.float32),
    )(inputs, featT, labrows, idx2d, acc, pick)


def kernel(inputs, indices, features, labels):
    labpad = jnp.pad(labels, (0, 782 * 128 - _N)).reshape(782, 128)
    featT = features.T
    idx2d = indices.reshape(_BATCH, 1)
    labrows = _sc_gather128(indices, labpad, 7, jnp.int32)
    acc, pick = _tc_main(inputs, featT, labrows, idx2d)
    loss = _tc_finish(inputs, featT, labrows, idx2d, acc, pick)
    return loss[0, 0]
